# baseline (speedup 1.0000x reference)
import jax
import jax.numpy as jnp
from jax import lax
from jax.experimental import pallas as pl
from jax.experimental.pallas import tpu as pltpu
from jax.experimental.pallas import tpu_sc as plsc

_D = 64
_B = 4096 * 200
_NW = 32
_RPW = _B // _NW
_CHUNK = 128
_NCH = _RPW // _CHUNK
_SCALE = 8.0


def _emb_body(idx_hbm, tab_hbm, out_hbm, idx_v, g0, g1, o0, o1,
              gs0, gs1, os0, os1):
    c = lax.axis_index("c")
    s = lax.axis_index("s")
    wid = s * 2 + c
    base = wid * _RPW

    gbuf = (g0, g1)
    obuf = (o0, o1)
    gsem = (gs0, gs1)
    osem = (os0, os1)

    pltpu.sync_copy(idx_hbm.at[wid], idx_v)

    def start_gather(g, b):
        pltpu.async_copy(tab_hbm.at[idx_v.at[g]], gbuf[b], gsem[b])

    def wait_gather(g, b):
        pltpu.make_async_copy(tab_hbm.at[idx_v.at[g]], gbuf[b], gsem[b]).wait()

    def start_out(g, b):
        pltpu.async_copy(
            obuf[b], out_hbm.at[pl.ds(base + g * _CHUNK, _CHUNK)], osem[b])

    def wait_out(g, b):
        pltpu.make_async_copy(
            obuf[b], out_hbm.at[pl.ds(base + g * _CHUNK, _CHUNK)],
            osem[b]).wait()

    def compute(g, b):
        def group(t, carry):
            iv = idx_v[g, pl.ds(t * 16, 16)]
            sc = jnp.where(iv != 0, jnp.float32(_SCALE), jnp.float32(0.0))
            for r in range(16):
                row = t * 16 + r
                srow = lax.broadcast(sc[r], (16,))
                for q in range(_D // 16):
                    obuf[b][row, pl.ds(q * 16, 16)] = (
                        gbuf[b][row, pl.ds(q * 16, 16)] * srow)
            return carry
        lax.fori_loop(0, _CHUNK // 16, group, 0)

    start_gather(0, 0)
    start_gather(1, 1)
    for g in (0, 1):
        b = g & 1
        wait_gather(g, b)
        compute(g, b)
        start_out(g, b)
        start_gather(g + 2, b)

    def steady(go, carry):
        for b in range(2):
            g = go * 2 + b
            wait_out(g - 2, b)
            wait_gather(g, b)
            compute(g, b)
            start_out(g, b)
            start_gather(g + 2, b)
        return carry

    lax.fori_loop(1, _NCH // 2 - 1, steady, 0)

    for g in (_NCH - 2, _NCH - 1):
        b = g & 1
        wait_out(g - 2, b)
        wait_gather(g, b)
        compute(g, b)
        start_out(g, b)
    for g in (_NCH - 2, _NCH - 1):
        wait_out(g, g & 1)


def kernel(input, lookup_table):
    idx = input.astype(jnp.int32).reshape(_NW, _NCH, _CHUNK)
    mesh = plsc.VectorSubcoreMesh(core_axis_name="c", subcore_axis_name="s")
    out = pl.kernel(
        _emb_body,
        out_type=jax.ShapeDtypeStruct((_B, _D), jnp.float32),
        mesh=mesh,
        compiler_params=pltpu.CompilerParams(use_tc_tiling_on_sc=False),
        scratch_types=[
            pltpu.VMEM((_NCH, _CHUNK), jnp.int32),
            pltpu.VMEM((_CHUNK, _D), jnp.float32),
            pltpu.VMEM((_CHUNK, _D), jnp.float32),
            pltpu.VMEM((_CHUNK, _D), jnp.float32),
            pltpu.VMEM((_CHUNK, _D), jnp.float32),
            pltpu.SemaphoreType.DMA,
            pltpu.SemaphoreType.DMA,
            pltpu.SemaphoreType.DMA,
            pltpu.SemaphoreType.DMA,
        ],
    )(idx, lookup_table)
    return out.reshape(input.shape[0], input.shape[1], _D)


# speedup vs baseline: 1.0130x; 1.0130x over previous
import jax
import jax.numpy as jnp
from jax import lax
from jax.experimental import pallas as pl
from jax.experimental.pallas import tpu as pltpu
from jax.experimental.pallas import tpu_sc as plsc

_BATCH = 4096
_TOK = 200
_D = 64
_NW = 32
_BPW = _BATCH // _NW
_S0 = 104
_S1 = _TOK - _S0
_NG = _TOK // 16
_SCALE = 8.0


def _emb_body(idx_hbm, tab_hbm, out_hbm, idx_v, g0, g1, o0, o1,
              gs0, gs1, os0, os1):
    c = lax.axis_index("c")
    s = lax.axis_index("s")
    wid = s * 2 + c
    row0 = wid * _BPW

    gbuf = (g0, g1)
    obuf = (o0, o1)
    gsem = (gs0, gs1)
    osem = (os0, os1)

    pltpu.sync_copy(idx_hbm.at[pl.ds(row0, _BPW)], idx_v)

    def start_gather(r, b):
        pltpu.async_copy(tab_hbm.at[idx_v.at[r, pl.ds(0, _S0)]],
                         gbuf[b].at[pl.ds(0, _S0)], gsem[b])
        pltpu.async_copy(tab_hbm.at[idx_v.at[r, pl.ds(_S0, _S1)]],
                         gbuf[b].at[pl.ds(_S0, _S1)], gsem[b])

    def wait_gather(r, b):
        pltpu.make_async_copy(tab_hbm.at[idx_v.at[r, pl.ds(0, _S0)]],
                              gbuf[b].at[pl.ds(0, _S0)], gsem[b]).wait()
        pltpu.make_async_copy(tab_hbm.at[idx_v.at[r, pl.ds(_S0, _S1)]],
                              gbuf[b].at[pl.ds(_S0, _S1)], gsem[b]).wait()

    def start_out(r, b):
        pltpu.async_copy(obuf[b], out_hbm.at[row0 + r], osem[b])

    def wait_out(r, b):
        pltpu.make_async_copy(obuf[b], out_hbm.at[row0 + r], osem[b]).wait()

    def scale_rows(iv, rows, lanes, b):
        sc = jnp.where(iv != 0, jnp.float32(_SCALE), jnp.float32(0.0))
        for i in range(len(rows)):
            srow = lax.broadcast(sc[lanes[i]], (16,))
            for q in range(_D // 16):
                obuf[b][rows[i], pl.ds(q * 16, 16)] = (
                    gbuf[b][rows[i], pl.ds(q * 16, 16)] * srow)

    def compute(r, b):
        def group(t, carry):
            iv = idx_v[r, pl.ds(t * 16, 16)]
            scale_rows(iv, [t * 16 + i for i in range(16)], list(range(16)), b)
            return carry
        lax.fori_loop(0, _NG, group, 0)
        iv = idx_v[r, pl.ds(_TOK - 16, 16)]
        scale_rows(iv, [_NG * 16 + i for i in range(8)],
                   [8 + i for i in range(8)], b)

    start_gather(0, 0)
    start_gather(1, 1)
    for r in (0, 1):
        b = r & 1
        wait_gather(r, b)
        compute(r, b)
        start_out(r, b)
        start_gather(r + 2, b)

    def steady(ro, carry):
        for b in range(2):
            r = ro * 2 + b
            wait_out(r - 2, b)
            wait_gather(r, b)
            compute(r, b)
            start_out(r, b)
            start_gather(r + 2, b)
        return carry

    lax.fori_loop(1, _BPW // 2 - 1, steady, 0)

    for r in (_BPW - 2, _BPW - 1):
        b = r & 1
        wait_out(r - 2, b)
        wait_gather(r, b)
        compute(r, b)
        start_out(r, b)
    for r in (_BPW - 2, _BPW - 1):
        wait_out(r, r & 1)


def kernel(input, lookup_table):
    idx = input.astype(jnp.int32)
    mesh = plsc.VectorSubcoreMesh(core_axis_name="c", subcore_axis_name="s")
    return pl.kernel(
        _emb_body,
        out_type=jax.ShapeDtypeStruct((_BATCH, _TOK, _D), jnp.float32),
        mesh=mesh,
        compiler_params=pltpu.CompilerParams(use_tc_tiling_on_sc=False),
        scratch_types=[
            pltpu.VMEM((_BPW, _TOK), jnp.int32),
            pltpu.VMEM((_TOK, _D), jnp.float32),
            pltpu.VMEM((_TOK, _D), jnp.float32),
            pltpu.VMEM((_TOK, _D), jnp.float32),
            pltpu.VMEM((_TOK, _D), jnp.float32),
            pltpu.SemaphoreType.DMA,
            pltpu.SemaphoreType.DMA,
            pltpu.SemaphoreType.DMA,
            pltpu.SemaphoreType.DMA,
        ],
    )(idx, lookup_table)
